# Initial kernel scaffold; baseline (speedup 1.0000x reference)
import functools

import jax
import jax.numpy as jnp
from jax import lax
from jax.experimental import pallas as pl
from jax.experimental.pallas import tpu as pltpu
from jax.experimental.pallas import tpu_sc as plsc

N = 10000
E = 320000
D = 128
NC = 2
NS = 16
NW = NC * NS
EPT = E // NW
C = 80
NCHUNK = EPT // C
RPS = N // NS
ZR = 125
LANES = 16
NVR = D // LANES

_mesh = plsc.VectorSubcoreMesh(core_axis_name="c", subcore_axis_name="s")


@functools.partial(
    pl.kernel,
    out_type=jax.ShapeDtypeStruct((NC * N, D), jnp.float32),
    mesh=_mesh,
    scratch_types=[
        pltpu.VMEM((C,), jnp.int32),
        pltpu.VMEM((C,), jnp.int32),
        pltpu.VMEM((C, D), jnp.float32),
        pltpu.VMEM((C, D), jnp.float32),
        pltpu.VMEM((ZR, D), jnp.float32),
        pltpu.VMEM_SHARED((N, D), jnp.float32),
        pltpu.SemaphoreType.DMA,
    ],
)
def _phase1(x_hbm, ea_hbm, src_hbm, dst_hbm, out_hbm,
            src_v, dst_v, xs_v, ea_v, zbuf, agg_sh, sem):
    cid = lax.axis_index("c")
    sid = lax.axis_index("s")
    wid = cid * NS + sid

    @pl.loop(0, ZR)
    def _(r):
        for j in range(NVR):
            zbuf[r, pl.ds(j * LANES, LANES)] = jnp.zeros((LANES,), jnp.float32)

    @pl.loop(0, RPS // ZR)
    def _(k):
        pltpu.sync_copy(zbuf, agg_sh.at[pl.ds(sid * RPS + k * ZR, ZR)])

    plsc.subcore_barrier()

    @pl.loop(0, NCHUNK)
    def _(i):
        base = wid * EPT + i * C
        pltpu.sync_copy(src_hbm.at[pl.ds(base, C)], src_v)
        pltpu.sync_copy(dst_hbm.at[pl.ds(base, C)], dst_v)
        pltpu.async_copy(x_hbm.at[src_v], xs_v, sem).wait()
        pltpu.sync_copy(ea_hbm.at[pl.ds(base, C)], ea_v)

        @pl.loop(0, C)
        def _(r):
            for j in range(NVR):
                sl = pl.ds(j * LANES, LANES)
                ea_v[r, sl] = ea_v[r, sl] * xs_v[r, sl]

        pltpu.sync_copy(ea_v, agg_sh.at[dst_v], add=True)

    plsc.subcore_barrier()

    pltpu.sync_copy(agg_sh.at[pl.ds(sid * RPS, RPS)],
                    out_hbm.at[pl.ds(cid * N + sid * RPS, RPS)])


@functools.partial(
    pl.kernel,
    out_type=(jax.ShapeDtypeStruct((E, D), jnp.float32),
              jax.ShapeDtypeStruct((E, D), jnp.float32)),
    mesh=_mesh,
    scratch_types=[
        pltpu.VMEM((C,), jnp.int32),
        pltpu.VMEM((C,), jnp.int32),
        pltpu.VMEM((C, D), jnp.float32),
        pltpu.VMEM((C, D), jnp.float32),
        pltpu.SemaphoreType.DMA,
        pltpu.SemaphoreType.DMA,
    ],
)
def _phase3(na_hbm, src_hbm, dst_hbm, gs_hbm, gd_hbm,
            src_v, dst_v, gs_v, gd_v, sem_a, sem_b):
    cid = lax.axis_index("c")
    sid = lax.axis_index("s")
    wid = cid * NS + sid

    @pl.loop(0, NCHUNK)
    def _(i):
        base = wid * EPT + i * C
        pltpu.sync_copy(src_hbm.at[pl.ds(base, C)], src_v)
        pltpu.sync_copy(dst_hbm.at[pl.ds(base, C)], dst_v)
        a = pltpu.async_copy(na_hbm.at[src_v], gs_v, sem_a)
        b = pltpu.async_copy(na_hbm.at[dst_v], gd_v, sem_b)
        a.wait()
        b.wait()
        pltpu.sync_copy(gs_v, gs_hbm.at[pl.ds(base, C)])
        pltpu.sync_copy(gd_v, gd_hbm.at[pl.ds(base, C)])


def _node_softmax_body(p_ref, x_ref, o_ref):
    a = p_ref[0:N, :] + p_ref[N:2 * N, :] + x_ref[...]
    m = jnp.max(a, axis=-1, keepdims=True)
    e = jnp.exp(a - m)
    o_ref[...] = e / jnp.sum(e, axis=-1, keepdims=True)


def _edge_softmax_body(gs_ref, gd_ref, ea_ref, o_ref):
    t = gs_ref[...] * gd_ref[...] + ea_ref[...]
    m = jnp.max(t, axis=-1, keepdims=True)
    e = jnp.exp(t - m)
    o_ref[...] = e / jnp.sum(e, axis=-1, keepdims=True)


_BE = 2000


def kernel(x, edge_attr, edge_index):
    src = edge_index[0].astype(jnp.int32)
    dst = edge_index[1].astype(jnp.int32)

    partials = _phase1(x, edge_attr, src, dst)

    node_att = pl.pallas_call(
        _node_softmax_body,
        out_shape=jax.ShapeDtypeStruct((N, D), jnp.float32),
    )(partials, x)

    gs, gd = _phase3(node_att, src, dst)

    edge_att_new = pl.pallas_call(
        _edge_softmax_body,
        grid=(E // _BE,),
        in_specs=[pl.BlockSpec((_BE, D), lambda i: (i, 0))] * 3,
        out_specs=pl.BlockSpec((_BE, D), lambda i: (i, 0)),
        out_shape=jax.ShapeDtypeStruct((E, D), jnp.float32),
    )(gs, gd, edge_attr)

    return node_att, edge_att_new


# speedup vs baseline: 3.0668x; 3.0668x over previous
import functools

import jax
import jax.numpy as jnp
from jax import lax
from jax.experimental import pallas as pl
from jax.experimental.pallas import tpu as pltpu
from jax.experimental.pallas import tpu_sc as plsc

N = 10000
E = 320000
D = 128
NC = 2
NS = 16
NW = NC * NS
EPT = E // NW
C = 80
NCHUNK = EPT // C
NPAD = 10240
RPS = NPAD // NS
ZR = 128
LANES = 16
NVR = D // LANES

_mesh = plsc.VectorSubcoreMesh(core_axis_name="c", subcore_axis_name="s")


@functools.partial(
    pl.kernel,
    out_type=jax.ShapeDtypeStruct((NC * NPAD, D), jnp.float32),
    mesh=_mesh,
    scratch_types=[
        pltpu.VMEM((C,), jnp.int32),
        pltpu.VMEM((C,), jnp.int32),
        pltpu.VMEM((C, D), jnp.float32),
        pltpu.VMEM((C, D), jnp.float32),
        pltpu.VMEM((ZR, D), jnp.float32),
        pltpu.VMEM_SHARED((NPAD, D), jnp.float32),
        pltpu.SemaphoreType.DMA,
    ],
)
def _phase1(x_hbm, ea_hbm, src_hbm, dst_hbm, out_hbm,
            src_v, dst_v, xs_v, ea_v, zbuf, agg_sh, sem):
    cid = lax.axis_index("c")
    sid = lax.axis_index("s")
    wid = cid * NS + sid

    @pl.loop(0, ZR)
    def _(r):
        for j in range(NVR):
            zbuf[r, pl.ds(j * LANES, LANES)] = jnp.zeros((LANES,), jnp.float32)

    @pl.loop(0, RPS // ZR)
    def _(k):
        pltpu.sync_copy(zbuf, agg_sh.at[pl.ds(sid * RPS + k * ZR, ZR)])

    plsc.subcore_barrier()

    @pl.loop(0, NCHUNK)
    def _(i):
        base = wid * EPT + i * C
        pltpu.sync_copy(src_hbm.at[pl.ds(base, C)], src_v)
        pltpu.sync_copy(dst_hbm.at[pl.ds(base, C)], dst_v)
        pltpu.async_copy(x_hbm.at[src_v], xs_v, sem).wait()
        pltpu.sync_copy(ea_hbm.at[pl.ds(base, C)], ea_v)

        @pl.loop(0, C)
        def _(r):
            for j in range(NVR):
                sl = pl.ds(j * LANES, LANES)
                ea_v[r, sl] = ea_v[r, sl] * xs_v[r, sl]

        pltpu.sync_copy(ea_v, agg_sh.at[dst_v], add=True)

    plsc.subcore_barrier()

    pltpu.sync_copy(agg_sh.at[pl.ds(sid * RPS, RPS)],
                    out_hbm.at[pl.ds(cid * NPAD + sid * RPS, RPS)])


@functools.partial(
    pl.kernel,
    out_type=(jax.ShapeDtypeStruct((E, D), jnp.float32),
              jax.ShapeDtypeStruct((E, D), jnp.float32)),
    mesh=_mesh,
    scratch_types=[
        pltpu.VMEM((C,), jnp.int32),
        pltpu.VMEM((C,), jnp.int32),
        pltpu.VMEM((C, D), jnp.float32),
        pltpu.VMEM((C, D), jnp.float32),
        pltpu.SemaphoreType.DMA,
        pltpu.SemaphoreType.DMA,
    ],
)
def _phase3(na_hbm, src_hbm, dst_hbm, gs_hbm, gd_hbm,
            src_v, dst_v, gs_v, gd_v, sem_a, sem_b):
    cid = lax.axis_index("c")
    sid = lax.axis_index("s")
    wid = cid * NS + sid

    @pl.loop(0, NCHUNK)
    def _(i):
        base = wid * EPT + i * C
        pltpu.sync_copy(src_hbm.at[pl.ds(base, C)], src_v)
        pltpu.sync_copy(dst_hbm.at[pl.ds(base, C)], dst_v)
        a = pltpu.async_copy(na_hbm.at[src_v], gs_v, sem_a)
        b = pltpu.async_copy(na_hbm.at[dst_v], gd_v, sem_b)
        a.wait()
        b.wait()
        pltpu.sync_copy(gs_v, gs_hbm.at[pl.ds(base, C)])
        pltpu.sync_copy(gd_v, gd_hbm.at[pl.ds(base, C)])


def _node_softmax_body(p_ref, x_ref, o_ref):
    a = p_ref[0:N, :] + p_ref[NPAD:NPAD + N, :] + x_ref[...]
    m = jnp.max(a, axis=-1, keepdims=True)
    e = jnp.exp(a - m)
    o_ref[...] = e / jnp.sum(e, axis=-1, keepdims=True)


def _edge_softmax_body(gs_ref, gd_ref, ea_ref, o_ref):
    t = gs_ref[...] * gd_ref[...] + ea_ref[...]
    m = jnp.max(t, axis=-1, keepdims=True)
    e = jnp.exp(t - m)
    o_ref[...] = e / jnp.sum(e, axis=-1, keepdims=True)


_BE = 2000


def kernel(x, edge_attr, edge_index):
    src = edge_index[0].astype(jnp.int32)
    dst = edge_index[1].astype(jnp.int32)

    partials = _phase1(x, edge_attr, src, dst)

    node_att = pl.pallas_call(
        _node_softmax_body,
        out_shape=jax.ShapeDtypeStruct((N, D), jnp.float32),
    )(partials, x)

    gs, gd = _phase3(node_att, src, dst)

    edge_att_new = pl.pallas_call(
        _edge_softmax_body,
        grid=(E // _BE,),
        in_specs=[pl.BlockSpec((_BE, D), lambda i: (i, 0))] * 3,
        out_specs=pl.BlockSpec((_BE, D), lambda i: (i, 0)),
        out_shape=jax.ShapeDtypeStruct((E, D), jnp.float32),
    )(gs, gd, edge_attr)

    return node_att, edge_att_new


# speedup vs baseline: 5.5231x; 1.8009x over previous
import functools

import jax
import jax.numpy as jnp
from jax import lax
from jax.experimental import pallas as pl
from jax.experimental.pallas import tpu as pltpu
from jax.experimental.pallas import tpu_sc as plsc

N = 10000
E = 320000
D = 128
NC = 2
NS = 16
NW = NC * NS
EPT = E // NW
C1 = 40
NCHUNK1 = EPT // C1
C3 = 80
NCHUNK3 = EPT // C3
NPAD = 10240
RPS = NPAD // NS
LANES = 16
NVR = D // LANES

_mesh = plsc.VectorSubcoreMesh(core_axis_name="c", subcore_axis_name="s")


def _mul_rows(nrows, acc, other):
    @pl.loop(0, nrows)
    def _(r):
        for j in range(NVR):
            sl = pl.ds(j * LANES, LANES)
            acc[r, sl] = acc[r, sl] * other[r, sl]


@functools.partial(
    pl.kernel,
    out_type=jax.ShapeDtypeStruct((NC * NPAD, D), jnp.float32),
    mesh=_mesh,
    scratch_types=[
        pltpu.VMEM((C1,), jnp.int32),
        pltpu.VMEM((C1,), jnp.int32),
        pltpu.VMEM((2, C1), jnp.int32),
        pltpu.VMEM((C1, D), jnp.float32),
        pltpu.VMEM((C1, D), jnp.float32),
        pltpu.VMEM((C1, D), jnp.float32),
        pltpu.VMEM((C1, D), jnp.float32),
        pltpu.VMEM_SHARED((NPAD, D), jnp.float32),
        pltpu.SemaphoreType.DMA,
        pltpu.SemaphoreType.DMA,
        pltpu.SemaphoreType.DMA,
        pltpu.SemaphoreType.DMA,
        pltpu.SemaphoreType.DMA,
        pltpu.SemaphoreType.DMA,
        pltpu.SemaphoreType.DMA,
        pltpu.SemaphoreType.DMA,
        pltpu.SemaphoreType.DMA,
        pltpu.SemaphoreType.DMA,
    ],
)
def _phase1(x_hbm, ea_hbm, src_hbm, dst_hbm, out_hbm,
            si0v, si1v, didx, xs0, xs1, ea0, ea1, agg_sh,
            sg0, sg1, se0, se1, ss0, ss1, si0, si1, sd0, sd1):
    cid = lax.axis_index("c")
    sid = lax.axis_index("s")
    wid = cid * NS + sid
    dslot0 = didx.at[0]
    dslot1 = didx.at[1]

    @pl.loop(0, C1)
    def _(r):
        for j in range(NVR):
            ea0[r, pl.ds(j * LANES, LANES)] = jnp.zeros((LANES,), jnp.float32)

    @pl.loop(0, RPS // C1)
    def _(k):
        pltpu.sync_copy(ea0, agg_sh.at[pl.ds(sid * RPS + k * C1, C1)])

    plsc.subcore_barrier()

    def issue_idx(j, siv, si):
        jc = jnp.minimum(j, NCHUNK1 - 1)
        pltpu.async_copy(src_hbm.at[wid, jc], siv, si)

    def wait_idx(siv, si):
        pltpu.make_async_copy(src_hbm.at[0, 0], siv, si).wait()

    def issue_didx(j, dslot, sd):
        jc = jnp.minimum(j, NCHUNK1 - 1)
        pltpu.async_copy(dst_hbm.at[wid, jc], dslot, sd)

    def wait_didx(dslot, sd):
        pltpu.make_async_copy(dst_hbm.at[0, 0], dslot, sd).wait()

    def issue(j, siv, xs, ea, sg, se):
        jc = jnp.minimum(j, NCHUNK1 - 1)
        pltpu.async_copy(x_hbm.at[siv], xs, sg)
        pltpu.async_copy(ea_hbm.at[pl.ds(wid * EPT + jc * C1, C1)], ea, se)

    def wait_loads(siv, xs, ea, sg, se):
        pltpu.make_async_copy(x_hbm.at[siv], xs, sg).wait()
        pltpu.make_async_copy(ea_hbm.at[pl.ds(0, C1)], ea, se).wait()

    def scatter(j, ea, dslot, ss):
        pltpu.async_copy(ea, agg_sh.at[dslot], ss, add=True)

    def wait_scatter(ea, ss):
        pltpu.make_async_copy(ea, agg_sh.at[dslot0], ss).wait()

    pltpu.sync_copy(dst_hbm.at[wid, 0], dslot0)
    pltpu.sync_copy(dst_hbm.at[wid, 1], dslot1)
    pltpu.sync_copy(src_hbm.at[wid, 0], si0v)
    pltpu.sync_copy(src_hbm.at[wid, 1], si1v)
    issue(0, si0v, xs0, ea0, sg0, se0)
    issue(1, si1v, xs1, ea1, sg1, se1)

    wait_loads(si0v, xs0, ea0, sg0, se0)
    issue_idx(2, si0v, si0)
    _mul_rows(C1, ea0, xs0)
    scatter(0, ea0, dslot0, ss0)

    wait_scatter(ea0, ss0)
    issue_didx(2, dslot0, sd0)
    wait_idx(si0v, si0)
    issue(2, si0v, xs0, ea0, sg0, se0)
    wait_loads(si1v, xs1, ea1, sg1, se1)
    issue_idx(3, si1v, si1)
    _mul_rows(C1, ea1, xs1)
    scatter(1, ea1, dslot1, ss1)

    @pl.loop(0, (NCHUNK1 - 2) // 2)
    def _(k):
        ja = 2 * k + 2
        wait_scatter(ea1, ss1)
        issue_didx(ja + 1, dslot1, sd1)
        wait_idx(si1v, si1)
        issue(ja + 1, si1v, xs1, ea1, sg1, se1)
        wait_loads(si0v, xs0, ea0, sg0, se0)
        issue_idx(ja + 2, si0v, si0)
        _mul_rows(C1, ea0, xs0)
        wait_didx(dslot0, sd0)
        scatter(ja, ea0, dslot0, ss0)

        jb = 2 * k + 3
        wait_scatter(ea0, ss0)
        issue_didx(jb + 1, dslot0, sd0)
        wait_idx(si0v, si0)
        issue(jb + 1, si0v, xs0, ea0, sg0, se0)
        wait_loads(si1v, xs1, ea1, sg1, se1)
        issue_idx(jb + 2, si1v, si1)
        _mul_rows(C1, ea1, xs1)
        wait_didx(dslot1, sd1)
        scatter(jb, ea1, dslot1, ss1)

    wait_scatter(ea1, ss1)
    wait_didx(dslot0, sd0)
    wait_idx(si1v, si1)
    wait_loads(si0v, xs0, ea0, sg0, se0)

    plsc.subcore_barrier()

    pltpu.sync_copy(agg_sh.at[pl.ds(sid * RPS, RPS)],
                    out_hbm.at[pl.ds(cid * NPAD + sid * RPS, RPS)])


@functools.partial(
    pl.kernel,
    out_type=jax.ShapeDtypeStruct((E, D), jnp.float32),
    mesh=_mesh,
    scratch_types=[
        pltpu.VMEM((NCHUNK3, C3), jnp.int32),
        pltpu.VMEM((NCHUNK3, C3), jnp.int32),
        pltpu.VMEM((C3, D), jnp.float32),
        pltpu.VMEM((C3, D), jnp.float32),
        pltpu.VMEM((C3, D), jnp.float32),
        pltpu.VMEM((C3, D), jnp.float32),
        pltpu.SemaphoreType.DMA,
        pltpu.SemaphoreType.DMA,
        pltpu.SemaphoreType.DMA,
        pltpu.SemaphoreType.DMA,
        pltpu.SemaphoreType.DMA,
        pltpu.SemaphoreType.DMA,
    ],
)
def _phase3(na_hbm, src_hbm, dst_hbm, prod_hbm,
            srci, dsti, gs0, gs1, gd0, gd1,
            sa0, sa1, sb0, sb1, sw0, sw1):
    cid = lax.axis_index("c")
    sid = lax.axis_index("s")
    wid = cid * NS + sid

    pltpu.sync_copy(src_hbm.at[wid], srci)
    pltpu.sync_copy(dst_hbm.at[wid], dsti)

    def issue(j, gs, gd, sa, sb):
        pltpu.async_copy(na_hbm.at[srci.at[j]], gs, sa)
        pltpu.async_copy(na_hbm.at[dsti.at[j]], gd, sb)

    def wait_loads(gs, gd, sa, sb):
        pltpu.make_async_copy(na_hbm.at[srci.at[0]], gs, sa).wait()
        pltpu.make_async_copy(na_hbm.at[dsti.at[0]], gd, sb).wait()

    def write(j, gs, sw):
        pltpu.async_copy(gs, prod_hbm.at[pl.ds(wid * EPT + j * C3, C3)], sw)

    def wait_write(gs, sw):
        pltpu.make_async_copy(gs, prod_hbm.at[pl.ds(0, C3)], sw).wait()

    issue(0, gs0, gd0, sa0, sb0)
    issue(1, gs1, gd1, sa1, sb1)
    wait_loads(gs0, gd0, sa0, sb0)
    _mul_rows(C3, gs0, gd0)
    write(0, gs0, sw0)

    @pl.loop(0, (NCHUNK3 - 3) // 2)
    def _(k):
        j1 = 2 * k + 1
        wait_write(gs0, sw0)
        issue(j1 + 1, gs0, gd0, sa0, sb0)
        wait_loads(gs1, gd1, sa1, sb1)
        _mul_rows(C3, gs1, gd1)
        write(j1, gs1, sw1)

        j2 = 2 * k + 2
        wait_write(gs1, sw1)
        issue(j2 + 1, gs1, gd1, sa1, sb1)
        wait_loads(gs0, gd0, sa0, sb0)
        _mul_rows(C3, gs0, gd0)
        write(j2, gs0, sw0)

    wait_write(gs0, sw0)
    issue(NCHUNK3 - 1, gs0, gd0, sa0, sb0)
    wait_loads(gs1, gd1, sa1, sb1)
    _mul_rows(C3, gs1, gd1)
    write(NCHUNK3 - 2, gs1, sw1)

    wait_loads(gs0, gd0, sa0, sb0)
    _mul_rows(C3, gs0, gd0)
    write(NCHUNK3 - 1, gs0, sw0)
    wait_write(gs0, sw0)
    wait_write(gs1, sw1)


def _node_softmax_body(p_ref, x_ref, o_ref):
    a = p_ref[0:N, :] + p_ref[NPAD:NPAD + N, :] + x_ref[...]
    m = jnp.max(a, axis=-1, keepdims=True)
    e = jnp.exp(a - m)
    o_ref[...] = e / jnp.sum(e, axis=-1, keepdims=True)


def _edge_softmax_body(prod_ref, ea_ref, o_ref):
    t = prod_ref[...] + ea_ref[...]
    m = jnp.max(t, axis=-1, keepdims=True)
    e = jnp.exp(t - m)
    o_ref[...] = e / jnp.sum(e, axis=-1, keepdims=True)


_BE = 2000


def kernel(x, edge_attr, edge_index):
    src = edge_index[0].astype(jnp.int32)
    dst = edge_index[1].astype(jnp.int32)

    partials = _phase1(x, edge_attr,
                       src.reshape(NW, NCHUNK1, C1),
                       dst.reshape(NW, NCHUNK1, C1))

    node_att = pl.pallas_call(
        _node_softmax_body,
        out_shape=jax.ShapeDtypeStruct((N, D), jnp.float32),
    )(partials, x)

    prod = _phase3(node_att,
                   src.reshape(NW, NCHUNK3, C3),
                   dst.reshape(NW, NCHUNK3, C3))

    edge_att_new = pl.pallas_call(
        _edge_softmax_body,
        grid=(E // _BE,),
        in_specs=[pl.BlockSpec((_BE, D), lambda i: (i, 0))] * 2,
        out_specs=pl.BlockSpec((_BE, D), lambda i: (i, 0)),
        out_shape=jax.ShapeDtypeStruct((E, D), jnp.float32),
    )(prod, edge_attr)

    return node_att, edge_att_new


# speedup vs baseline: 5.9483x; 1.0770x over previous
import functools

import jax
import jax.numpy as jnp
from jax import lax
from jax.experimental import pallas as pl
from jax.experimental.pallas import tpu as pltpu
from jax.experimental.pallas import tpu_sc as plsc

N = 10000
E = 320000
D = 128
NC = 2
NS = 16
NW = NC * NS
EPT = E // NW
C1 = 40
NCHUNK1 = EPT // C1
C3 = 80
KS = 5
ES = E // KS
EPTS = ES // NW
NCHUNK3 = EPTS // C3
NPAD = 10240
RPS = NPAD // NS
LANES = 16
NVR = D // LANES

_mesh = plsc.VectorSubcoreMesh(core_axis_name="c", subcore_axis_name="s")


def _mul_rows(nrows, acc, other):
    @pl.loop(0, nrows)
    def _(r):
        for j in range(NVR):
            sl = pl.ds(j * LANES, LANES)
            acc[r, sl] = acc[r, sl] * other[r, sl]


@functools.partial(
    pl.kernel,
    out_type=jax.ShapeDtypeStruct((NC * NPAD, D), jnp.float32),
    mesh=_mesh,
    scratch_types=[
        pltpu.VMEM((C1,), jnp.int32),
        pltpu.VMEM((C1,), jnp.int32),
        pltpu.VMEM((2, C1), jnp.int32),
        pltpu.VMEM((C1, D), jnp.float32),
        pltpu.VMEM((C1, D), jnp.float32),
        pltpu.VMEM((C1, D), jnp.float32),
        pltpu.VMEM((C1, D), jnp.float32),
        pltpu.VMEM_SHARED((NPAD, D), jnp.float32),
        pltpu.SemaphoreType.DMA,
        pltpu.SemaphoreType.DMA,
        pltpu.SemaphoreType.DMA,
        pltpu.SemaphoreType.DMA,
        pltpu.SemaphoreType.DMA,
        pltpu.SemaphoreType.DMA,
        pltpu.SemaphoreType.DMA,
        pltpu.SemaphoreType.DMA,
        pltpu.SemaphoreType.DMA,
        pltpu.SemaphoreType.DMA,
    ],
)
def _phase1(x_hbm, ea_hbm, src_hbm, dst_hbm, out_hbm,
            si0v, si1v, didx, xs0, xs1, ea0, ea1, agg_sh,
            sg0, sg1, se0, se1, ss0, ss1, si0, si1, sd0, sd1):
    cid = lax.axis_index("c")
    sid = lax.axis_index("s")
    wid = cid * NS + sid
    dslot0 = didx.at[0]
    dslot1 = didx.at[1]

    @pl.loop(0, C1)
    def _(r):
        for j in range(NVR):
            ea0[r, pl.ds(j * LANES, LANES)] = jnp.zeros((LANES,), jnp.float32)

    @pl.loop(0, RPS // C1)
    def _(k):
        pltpu.sync_copy(ea0, agg_sh.at[pl.ds(sid * RPS + k * C1, C1)])

    plsc.subcore_barrier()

    def issue_idx(j, siv, si):
        jc = jnp.minimum(j, NCHUNK1 - 1)
        pltpu.async_copy(src_hbm.at[wid, jc], siv, si)

    def wait_idx(siv, si):
        pltpu.make_async_copy(src_hbm.at[0, 0], siv, si).wait()

    def issue_didx(j, dslot, sd):
        jc = jnp.minimum(j, NCHUNK1 - 1)
        pltpu.async_copy(dst_hbm.at[wid, jc], dslot, sd)

    def wait_didx(dslot, sd):
        pltpu.make_async_copy(dst_hbm.at[0, 0], dslot, sd).wait()

    def issue(j, siv, xs, ea, sg, se):
        jc = jnp.minimum(j, NCHUNK1 - 1)
        pltpu.async_copy(x_hbm.at[siv], xs, sg)
        pltpu.async_copy(ea_hbm.at[pl.ds(wid * EPT + jc * C1, C1)], ea, se)

    def wait_loads(siv, xs, ea, sg, se):
        pltpu.make_async_copy(x_hbm.at[siv], xs, sg).wait()
        pltpu.make_async_copy(ea_hbm.at[pl.ds(0, C1)], ea, se).wait()

    def scatter(j, ea, dslot, ss):
        pltpu.async_copy(ea, agg_sh.at[dslot], ss, add=True)

    def wait_scatter(ea, ss):
        pltpu.make_async_copy(ea, agg_sh.at[dslot0], ss).wait()

    pltpu.sync_copy(dst_hbm.at[wid, 0], dslot0)
    pltpu.sync_copy(dst_hbm.at[wid, 1], dslot1)
    pltpu.sync_copy(src_hbm.at[wid, 0], si0v)
    pltpu.sync_copy(src_hbm.at[wid, 1], si1v)
    issue(0, si0v, xs0, ea0, sg0, se0)
    issue(1, si1v, xs1, ea1, sg1, se1)

    wait_loads(si0v, xs0, ea0, sg0, se0)
    issue_idx(2, si0v, si0)
    _mul_rows(C1, ea0, xs0)
    scatter(0, ea0, dslot0, ss0)

    wait_scatter(ea0, ss0)
    issue_didx(2, dslot0, sd0)
    wait_idx(si0v, si0)
    issue(2, si0v, xs0, ea0, sg0, se0)
    wait_loads(si1v, xs1, ea1, sg1, se1)
    issue_idx(3, si1v, si1)
    _mul_rows(C1, ea1, xs1)
    scatter(1, ea1, dslot1, ss1)

    @pl.loop(0, (NCHUNK1 - 2) // 2)
    def _(k):
        ja = 2 * k + 2
        wait_scatter(ea1, ss1)
        issue_didx(ja + 1, dslot1, sd1)
        wait_idx(si1v, si1)
        issue(ja + 1, si1v, xs1, ea1, sg1, se1)
        wait_loads(si0v, xs0, ea0, sg0, se0)
        issue_idx(ja + 2, si0v, si0)
        _mul_rows(C1, ea0, xs0)
        wait_didx(dslot0, sd0)
        scatter(ja, ea0, dslot0, ss0)

        jb = 2 * k + 3
        wait_scatter(ea0, ss0)
        issue_didx(jb + 1, dslot0, sd0)
        wait_idx(si0v, si0)
        issue(jb + 1, si0v, xs0, ea0, sg0, se0)
        wait_loads(si1v, xs1, ea1, sg1, se1)
        issue_idx(jb + 2, si1v, si1)
        _mul_rows(C1, ea1, xs1)
        wait_didx(dslot1, sd1)
        scatter(jb, ea1, dslot1, ss1)

    wait_scatter(ea1, ss1)
    wait_didx(dslot0, sd0)
    wait_idx(si1v, si1)
    wait_loads(si0v, xs0, ea0, sg0, se0)

    plsc.subcore_barrier()

    pltpu.sync_copy(agg_sh.at[pl.ds(sid * RPS, RPS)],
                    out_hbm.at[pl.ds(cid * NPAD + sid * RPS, RPS)])


def _make_phase3(ks):
    return functools.partial(
        pl.kernel,
        out_type=jax.ShapeDtypeStruct((ES, D), jnp.float32),
        mesh=_mesh,
        scratch_types=[
            pltpu.VMEM((NCHUNK3, C3), jnp.int32),
            pltpu.VMEM((NCHUNK3, C3), jnp.int32),
            pltpu.VMEM((C3, D), jnp.float32),
            pltpu.VMEM((C3, D), jnp.float32),
            pltpu.VMEM((C3, D), jnp.float32),
            pltpu.VMEM((C3, D), jnp.float32),
            pltpu.SemaphoreType.DMA,
            pltpu.SemaphoreType.DMA,
            pltpu.SemaphoreType.DMA,
            pltpu.SemaphoreType.DMA,
            pltpu.SemaphoreType.DMA,
            pltpu.SemaphoreType.DMA,
        ],
    )(functools.partial(_phase3_body, ks))


def _phase3_body(ks, na_hbm, src_hbm, dst_hbm, prod_hbm,
                 srci, dsti, gs0, gs1, gd0, gd1,
                 sa0, sa1, sb0, sb1, sw0, sw1):
    cid = lax.axis_index("c")
    sid = lax.axis_index("s")
    wid = cid * NS + sid

    pltpu.sync_copy(src_hbm.at[ks, wid], srci)
    pltpu.sync_copy(dst_hbm.at[ks, wid], dsti)

    def issue(j, gs, gd, sa, sb):
        pltpu.async_copy(na_hbm.at[srci.at[j]], gs, sa)
        pltpu.async_copy(na_hbm.at[dsti.at[j]], gd, sb)

    def wait_loads(gs, gd, sa, sb):
        pltpu.make_async_copy(na_hbm.at[srci.at[0]], gs, sa).wait()
        pltpu.make_async_copy(na_hbm.at[dsti.at[0]], gd, sb).wait()

    def write(j, gs, sw):
        pltpu.async_copy(gs, prod_hbm.at[pl.ds(wid * EPTS + j * C3, C3)], sw)

    def wait_write(gs, sw):
        pltpu.make_async_copy(gs, prod_hbm.at[pl.ds(0, C3)], sw).wait()

    issue(0, gs0, gd0, sa0, sb0)
    issue(1, gs1, gd1, sa1, sb1)
    wait_loads(gs0, gd0, sa0, sb0)
    _mul_rows(C3, gs0, gd0)
    write(0, gs0, sw0)

    @pl.loop(0, (NCHUNK3 - 3) // 2)
    def _(k):
        j1 = 2 * k + 1
        wait_write(gs0, sw0)
        issue(j1 + 1, gs0, gd0, sa0, sb0)
        wait_loads(gs1, gd1, sa1, sb1)
        _mul_rows(C3, gs1, gd1)
        write(j1, gs1, sw1)

        j2 = 2 * k + 2
        wait_write(gs1, sw1)
        issue(j2 + 1, gs1, gd1, sa1, sb1)
        wait_loads(gs0, gd0, sa0, sb0)
        _mul_rows(C3, gs0, gd0)
        write(j2, gs0, sw0)

    wait_write(gs0, sw0)
    issue(NCHUNK3 - 1, gs0, gd0, sa0, sb0)
    wait_loads(gs1, gd1, sa1, sb1)
    _mul_rows(C3, gs1, gd1)
    write(NCHUNK3 - 2, gs1, sw1)

    wait_loads(gs0, gd0, sa0, sb0)
    _mul_rows(C3, gs0, gd0)
    write(NCHUNK3 - 1, gs0, sw0)
    wait_write(gs0, sw0)
    wait_write(gs1, sw1)


def _node_softmax_body(p_ref, x_ref, o_ref):
    a = p_ref[0:N, :] + p_ref[NPAD:NPAD + N, :] + x_ref[...]
    m = jnp.max(a, axis=-1, keepdims=True)
    e = jnp.exp(a - m)
    o_ref[...] = e / jnp.sum(e, axis=-1, keepdims=True)


def _edge_softmax_first(prod_ref, ea_ref, o_ref):
    t = prod_ref[...] + ea_ref[...]
    m = jnp.max(t, axis=-1, keepdims=True)
    e = jnp.exp(t - m)
    o_ref[...] = e / jnp.sum(e, axis=-1, keepdims=True)


def _edge_softmax_next(buf_ref, prod_ref, ea_ref, o_ref):
    del buf_ref
    t = prod_ref[...] + ea_ref[...]
    m = jnp.max(t, axis=-1, keepdims=True)
    e = jnp.exp(t - m)
    o_ref[...] = e / jnp.sum(e, axis=-1, keepdims=True)


_BE = 2000
_BPS = ES // _BE

_PHASE3 = [_make_phase3(k) for k in range(KS)]


def kernel(x, edge_attr, edge_index):
    src = edge_index[0].astype(jnp.int32)
    dst = edge_index[1].astype(jnp.int32)

    partials = _phase1(x, edge_attr,
                       src.reshape(NW, NCHUNK1, C1),
                       dst.reshape(NW, NCHUNK1, C1))

    node_att = pl.pallas_call(
        _node_softmax_body,
        out_shape=jax.ShapeDtypeStruct((N, D), jnp.float32),
    )(partials, x)

    src3 = src.reshape(KS, NW, NCHUNK3, C3)
    dst3 = dst.reshape(KS, NW, NCHUNK3, C3)

    prods = [_PHASE3[k](node_att, src3, dst3) for k in range(KS)]

    edge_att_new = pl.pallas_call(
        _edge_softmax_first,
        grid=(_BPS,),
        in_specs=[pl.BlockSpec((_BE, D), lambda i: (i, 0))] * 2,
        out_specs=pl.BlockSpec((_BE, D), lambda i: (i, 0)),
        out_shape=jax.ShapeDtypeStruct((E, D), jnp.float32),
    )(prods[0], edge_attr)

    for k in range(1, KS):
        off = k * _BPS
        edge_att_new = pl.pallas_call(
            _edge_softmax_next,
            grid=(_BPS,),
            in_specs=[
                pl.BlockSpec(memory_space=pl.ANY),
                pl.BlockSpec((_BE, D), lambda i: (i, 0)),
                pl.BlockSpec((_BE, D), lambda i, off=off: (i + off, 0)),
            ],
            out_specs=pl.BlockSpec((_BE, D), lambda i, off=off: (i + off, 0)),
            out_shape=jax.ShapeDtypeStruct((E, D), jnp.float32),
            input_output_aliases={0: 0},
        )(edge_att_new, prods[k], edge_attr)

    return node_att, edge_att_new


# speedup vs baseline: 6.1861x; 1.0400x over previous
import functools

import jax
import jax.numpy as jnp
from jax import lax
from jax.experimental import pallas as pl
from jax.experimental.pallas import tpu as pltpu
from jax.experimental.pallas import tpu_sc as plsc

N = 10000
E = 320000
D = 128
NC = 2
NS = 16
NW = NC * NS
EPT = E // NW
C1 = 40
NCHUNK1 = EPT // C1
C3 = 40
KS = 2
ES = E // KS
EPTS = ES // NW
NCHUNK3 = EPTS // C3
NPAD = 10240
RPS = NPAD // NS
LANES = 16
NVR = D // LANES

_mesh = plsc.VectorSubcoreMesh(core_axis_name="c", subcore_axis_name="s")


def _mul_rows(nrows, acc, other):
    @pl.loop(0, nrows)
    def _(r):
        for j in range(NVR):
            sl = pl.ds(j * LANES, LANES)
            acc[r, sl] = acc[r, sl] * other[r, sl]


@functools.partial(
    pl.kernel,
    out_type=jax.ShapeDtypeStruct((NC * NPAD, D), jnp.float32),
    mesh=_mesh,
    scratch_types=[
        pltpu.VMEM((C1,), jnp.int32),
        pltpu.VMEM((C1,), jnp.int32),
        pltpu.VMEM((2, C1), jnp.int32),
        pltpu.VMEM((C1, D), jnp.float32),
        pltpu.VMEM((C1, D), jnp.float32),
        pltpu.VMEM((C1, D), jnp.float32),
        pltpu.VMEM((C1, D), jnp.float32),
        pltpu.VMEM_SHARED((NPAD, D), jnp.float32),
        pltpu.SemaphoreType.DMA,
        pltpu.SemaphoreType.DMA,
        pltpu.SemaphoreType.DMA,
        pltpu.SemaphoreType.DMA,
        pltpu.SemaphoreType.DMA,
        pltpu.SemaphoreType.DMA,
        pltpu.SemaphoreType.DMA,
        pltpu.SemaphoreType.DMA,
        pltpu.SemaphoreType.DMA,
        pltpu.SemaphoreType.DMA,
    ],
)
def _phase1(x_hbm, ea_hbm, src_hbm, dst_hbm, out_hbm,
            si0v, si1v, didx, xs0, xs1, ea0, ea1, agg_sh,
            sg0, sg1, se0, se1, ss0, ss1, si0, si1, sd0, sd1):
    cid = lax.axis_index("c")
    sid = lax.axis_index("s")
    wid = cid * NS + sid
    dslot0 = didx.at[0]
    dslot1 = didx.at[1]

    @pl.loop(0, C1)
    def _(r):
        for j in range(NVR):
            ea0[r, pl.ds(j * LANES, LANES)] = jnp.zeros((LANES,), jnp.float32)

    @pl.loop(0, RPS // C1)
    def _(k):
        pltpu.sync_copy(ea0, agg_sh.at[pl.ds(sid * RPS + k * C1, C1)])

    plsc.subcore_barrier()

    def issue_idx(j, siv, si):
        jc = jnp.minimum(j, NCHUNK1 - 1)
        pltpu.async_copy(src_hbm.at[wid, jc], siv, si)

    def wait_idx(siv, si):
        pltpu.make_async_copy(src_hbm.at[0, 0], siv, si).wait()

    def issue_didx(j, dslot, sd):
        jc = jnp.minimum(j, NCHUNK1 - 1)
        pltpu.async_copy(dst_hbm.at[wid, jc], dslot, sd)

    def wait_didx(dslot, sd):
        pltpu.make_async_copy(dst_hbm.at[0, 0], dslot, sd).wait()

    def issue(j, siv, xs, ea, sg, se):
        jc = jnp.minimum(j, NCHUNK1 - 1)
        pltpu.async_copy(x_hbm.at[siv], xs, sg)
        pltpu.async_copy(ea_hbm.at[pl.ds(wid * EPT + jc * C1, C1)], ea, se)

    def wait_loads(siv, xs, ea, sg, se):
        pltpu.make_async_copy(x_hbm.at[siv], xs, sg).wait()
        pltpu.make_async_copy(ea_hbm.at[pl.ds(0, C1)], ea, se).wait()

    def scatter(j, ea, dslot, ss):
        pltpu.async_copy(ea, agg_sh.at[dslot], ss, add=True)

    def wait_scatter(ea, ss):
        pltpu.make_async_copy(ea, agg_sh.at[dslot0], ss).wait()

    pltpu.sync_copy(dst_hbm.at[wid, 0], dslot0)
    pltpu.sync_copy(dst_hbm.at[wid, 1], dslot1)
    pltpu.sync_copy(src_hbm.at[wid, 0], si0v)
    pltpu.sync_copy(src_hbm.at[wid, 1], si1v)
    issue(0, si0v, xs0, ea0, sg0, se0)
    issue(1, si1v, xs1, ea1, sg1, se1)

    wait_loads(si0v, xs0, ea0, sg0, se0)
    issue_idx(2, si0v, si0)
    _mul_rows(C1, ea0, xs0)
    scatter(0, ea0, dslot0, ss0)

    wait_scatter(ea0, ss0)
    issue_didx(2, dslot0, sd0)
    wait_idx(si0v, si0)
    issue(2, si0v, xs0, ea0, sg0, se0)
    wait_loads(si1v, xs1, ea1, sg1, se1)
    issue_idx(3, si1v, si1)
    _mul_rows(C1, ea1, xs1)
    scatter(1, ea1, dslot1, ss1)

    @pl.loop(0, (NCHUNK1 - 2) // 2)
    def _(k):
        ja = 2 * k + 2
        wait_scatter(ea1, ss1)
        issue_didx(ja + 1, dslot1, sd1)
        wait_idx(si1v, si1)
        issue(ja + 1, si1v, xs1, ea1, sg1, se1)
        wait_loads(si0v, xs0, ea0, sg0, se0)
        issue_idx(ja + 2, si0v, si0)
        _mul_rows(C1, ea0, xs0)
        wait_didx(dslot0, sd0)
        scatter(ja, ea0, dslot0, ss0)

        jb = 2 * k + 3
        wait_scatter(ea0, ss0)
        issue_didx(jb + 1, dslot0, sd0)
        wait_idx(si0v, si0)
        issue(jb + 1, si0v, xs0, ea0, sg0, se0)
        wait_loads(si1v, xs1, ea1, sg1, se1)
        issue_idx(jb + 2, si1v, si1)
        _mul_rows(C1, ea1, xs1)
        wait_didx(dslot1, sd1)
        scatter(jb, ea1, dslot1, ss1)

    wait_scatter(ea1, ss1)
    wait_didx(dslot0, sd0)
    wait_idx(si1v, si1)
    wait_loads(si0v, xs0, ea0, sg0, se0)

    plsc.subcore_barrier()

    pltpu.sync_copy(agg_sh.at[pl.ds(sid * RPS, RPS)],
                    out_hbm.at[pl.ds(cid * NPAD + sid * RPS, RPS)])


def _make_phase3(ks):
    return functools.partial(
        pl.kernel,
        out_type=jax.ShapeDtypeStruct((ES, D), jnp.float32),
        mesh=_mesh,
        scratch_types=[
            pltpu.VMEM((C3,), jnp.int32),
            pltpu.VMEM((C3,), jnp.int32),
            pltpu.VMEM((C3,), jnp.int32),
            pltpu.VMEM((C3,), jnp.int32),
            pltpu.VMEM((C3, D), jnp.float32),
            pltpu.VMEM((C3, D), jnp.float32),
            pltpu.VMEM((C3, D), jnp.float32),
            pltpu.VMEM((C3, D), jnp.float32),
            pltpu.VMEM_SHARED((NPAD, D), jnp.float32),
            pltpu.SemaphoreType.DMA,
            pltpu.SemaphoreType.DMA,
            pltpu.SemaphoreType.DMA,
            pltpu.SemaphoreType.DMA,
            pltpu.SemaphoreType.DMA,
            pltpu.SemaphoreType.DMA,
            pltpu.SemaphoreType.DMA,
            pltpu.SemaphoreType.DMA,
            pltpu.SemaphoreType.DMA,
            pltpu.SemaphoreType.DMA,
        ],
    )(functools.partial(_phase3_body, ks))


def _phase3_body(ks, na_hbm, src_hbm, dst_hbm, prod_hbm,
                 siv0, siv1, div0, div1, gs0, gs1, gd0, gd1, na_sh,
                 sg0, sg1, sdg0, sdg1, sw0, sw1, si0, si1, sdi0, sdi1):
    cid = lax.axis_index("c")
    sid = lax.axis_index("s")
    wid = cid * NS + sid

    @pl.when(sid < NS - 1)
    def _():
        pltpu.sync_copy(na_hbm.at[pl.ds(sid * RPS, RPS)],
                        na_sh.at[pl.ds(sid * RPS, RPS)])

    @pl.when(sid == NS - 1)
    def _():
        pltpu.sync_copy(na_hbm.at[pl.ds((NS - 1) * RPS, N - (NS - 1) * RPS)],
                        na_sh.at[pl.ds((NS - 1) * RPS, N - (NS - 1) * RPS)])

    def issue_idx(j, siv, div, si, sdi):
        jc = jnp.minimum(j, NCHUNK3 - 1)
        pltpu.async_copy(src_hbm.at[ks, wid, jc], siv, si)
        pltpu.async_copy(dst_hbm.at[ks, wid, jc], div, sdi)

    def wait_idx(siv, div, si, sdi):
        pltpu.make_async_copy(src_hbm.at[0, 0, 0], siv, si).wait()
        pltpu.make_async_copy(dst_hbm.at[0, 0, 0], div, sdi).wait()

    def issue(siv, div, gs, gd, sg, sdg):
        pltpu.async_copy(na_sh.at[siv], gs, sg)
        pltpu.async_copy(na_sh.at[div], gd, sdg)

    def wait_gathers(siv, div, gs, gd, sg, sdg):
        pltpu.make_async_copy(na_sh.at[siv], gs, sg).wait()
        pltpu.make_async_copy(na_sh.at[div], gd, sdg).wait()

    def write(j, gs, sw):
        pltpu.async_copy(gs, prod_hbm.at[pl.ds(wid * EPTS + j * C3, C3)], sw)

    def wait_write(gs, sw):
        pltpu.make_async_copy(gs, prod_hbm.at[pl.ds(0, C3)], sw).wait()

    pltpu.sync_copy(src_hbm.at[ks, wid, 0], siv0)
    pltpu.sync_copy(dst_hbm.at[ks, wid, 0], div0)
    pltpu.sync_copy(src_hbm.at[ks, wid, 1], siv1)
    pltpu.sync_copy(dst_hbm.at[ks, wid, 1], div1)
    plsc.subcore_barrier()

    issue(siv0, div0, gs0, gd0, sg0, sdg0)
    issue(siv1, div1, gs1, gd1, sg1, sdg1)

    wait_gathers(siv0, div0, gs0, gd0, sg0, sdg0)
    issue_idx(2, siv0, div0, si0, sdi0)
    _mul_rows(C3, gs0, gd0)
    write(0, gs0, sw0)

    wait_write(gs0, sw0)
    wait_idx(siv0, div0, si0, sdi0)
    issue(siv0, div0, gs0, gd0, sg0, sdg0)
    wait_gathers(siv1, div1, gs1, gd1, sg1, sdg1)
    issue_idx(3, siv1, div1, si1, sdi1)
    _mul_rows(C3, gs1, gd1)
    write(1, gs1, sw1)

    @pl.loop(0, (NCHUNK3 - 3) // 2)
    def _(k):
        ja = 2 * k + 2
        wait_write(gs1, sw1)
        wait_idx(siv1, div1, si1, sdi1)
        issue(siv1, div1, gs1, gd1, sg1, sdg1)
        wait_gathers(siv0, div0, gs0, gd0, sg0, sdg0)
        issue_idx(ja + 2, siv0, div0, si0, sdi0)
        _mul_rows(C3, gs0, gd0)
        write(ja, gs0, sw0)

        jb = 2 * k + 3
        wait_write(gs0, sw0)
        wait_idx(siv0, div0, si0, sdi0)
        issue(siv0, div0, gs0, gd0, sg0, sdg0)
        wait_gathers(siv1, div1, gs1, gd1, sg1, sdg1)
        issue_idx(jb + 2, siv1, div1, si1, sdi1)
        _mul_rows(C3, gs1, gd1)
        write(jb, gs1, sw1)

    wait_write(gs1, sw1)
    wait_gathers(siv0, div0, gs0, gd0, sg0, sdg0)
    _mul_rows(C3, gs0, gd0)
    write(NCHUNK3 - 1, gs0, sw0)
    wait_write(gs0, sw0)
    wait_idx(siv1, div1, si1, sdi1)


def _node_softmax_body(p_ref, x_ref, o_ref):
    a = p_ref[0:N, :] + p_ref[NPAD:NPAD + N, :] + x_ref[...]
    m = jnp.max(a, axis=-1, keepdims=True)
    e = jnp.exp(a - m)
    o_ref[...] = e / jnp.sum(e, axis=-1, keepdims=True)


def _edge_softmax_first(prod_ref, ea_ref, o_ref):
    t = prod_ref[...] + ea_ref[...]
    m = jnp.max(t, axis=-1, keepdims=True)
    e = jnp.exp(t - m)
    o_ref[...] = e / jnp.sum(e, axis=-1, keepdims=True)


def _edge_softmax_next(buf_ref, prod_ref, ea_ref, o_ref):
    del buf_ref
    t = prod_ref[...] + ea_ref[...]
    m = jnp.max(t, axis=-1, keepdims=True)
    e = jnp.exp(t - m)
    o_ref[...] = e / jnp.sum(e, axis=-1, keepdims=True)


_BE = 2000
_BPS = ES // _BE

_PHASE3 = [_make_phase3(k) for k in range(KS)]


def kernel(x, edge_attr, edge_index):
    src = edge_index[0].astype(jnp.int32)
    dst = edge_index[1].astype(jnp.int32)

    partials = _phase1(x, edge_attr,
                       src.reshape(NW, NCHUNK1, C1),
                       dst.reshape(NW, NCHUNK1, C1))

    node_att = pl.pallas_call(
        _node_softmax_body,
        out_shape=jax.ShapeDtypeStruct((N, D), jnp.float32),
    )(partials, x)

    src3 = src.reshape(KS, NW, NCHUNK3, C3)
    dst3 = dst.reshape(KS, NW, NCHUNK3, C3)

    prods = [_PHASE3[k](node_att, src3, dst3) for k in range(KS)]

    edge_att_new = pl.pallas_call(
        _edge_softmax_first,
        grid=(_BPS,),
        in_specs=[pl.BlockSpec((_BE, D), lambda i: (i, 0))] * 2,
        out_specs=pl.BlockSpec((_BE, D), lambda i: (i, 0)),
        out_shape=jax.ShapeDtypeStruct((E, D), jnp.float32),
    )(prods[0], edge_attr)

    for k in range(1, KS):
        off = k * _BPS
        edge_att_new = pl.pallas_call(
            _edge_softmax_next,
            grid=(_BPS,),
            in_specs=[
                pl.BlockSpec(memory_space=pl.ANY),
                pl.BlockSpec((_BE, D), lambda i: (i, 0)),
                pl.BlockSpec((_BE, D), lambda i, off=off: (i + off, 0)),
            ],
            out_specs=pl.BlockSpec((_BE, D), lambda i, off=off: (i + off, 0)),
            out_shape=jax.ShapeDtypeStruct((E, D), jnp.float32),
            input_output_aliases={0: 0},
        )(edge_att_new, prods[k], edge_attr)

    return node_att, edge_att_new


# speedup vs baseline: 6.5155x; 1.0533x over previous
import functools

import jax
import jax.numpy as jnp
from jax import lax
from jax.experimental import pallas as pl
from jax.experimental.pallas import tpu as pltpu
from jax.experimental.pallas import tpu_sc as plsc

N = 10000
E = 320000
D = 128
NC = 2
NS = 16
NW = NC * NS
EPT = E // NW
C1 = 64
NF1 = EPT // C1
CT1 = EPT - NF1 * C1
C3 = 40
KS = 2
ES = E // KS
EPTS = ES // NW
NCHUNK3 = EPTS // C3
NPAD = 10240
RPS = NPAD // NS
LANES = 16
NVR = D // LANES

_mesh = plsc.VectorSubcoreMesh(core_axis_name="c", subcore_axis_name="s")


def _mul_rows(nrows, acc, other):
    @pl.loop(0, nrows)
    def _(r):
        for j in range(NVR):
            sl = pl.ds(j * LANES, LANES)
            acc[r, sl] = acc[r, sl] * other[r, sl]


@functools.partial(
    pl.kernel,
    out_type=jax.ShapeDtypeStruct((NC * NPAD, D), jnp.float32),
    mesh=_mesh,
    scratch_types=[
        pltpu.VMEM((C1,), jnp.int32),
        pltpu.VMEM((C1,), jnp.int32),
        pltpu.VMEM((2, C1), jnp.int32),
        pltpu.VMEM((1, CT1), jnp.int32),
        pltpu.VMEM((C1, D), jnp.float32),
        pltpu.VMEM((C1, D), jnp.float32),
        pltpu.VMEM((C1, D), jnp.float32),
        pltpu.VMEM((C1, D), jnp.float32),
        pltpu.VMEM_SHARED((NPAD, D), jnp.float32),
        pltpu.SemaphoreType.DMA,
        pltpu.SemaphoreType.DMA,
        pltpu.SemaphoreType.DMA,
        pltpu.SemaphoreType.DMA,
        pltpu.SemaphoreType.DMA,
        pltpu.SemaphoreType.DMA,
        pltpu.SemaphoreType.DMA,
        pltpu.SemaphoreType.DMA,
        pltpu.SemaphoreType.DMA,
        pltpu.SemaphoreType.DMA,
    ],
)
def _phase1(x_hbm, ea_hbm, src_hbm, dst_hbm, out_hbm,
            si0v, si1v, didx, didx_t, xs0, xs1, ea0, ea1, agg_sh,
            sg0, sg1, se0, se1, ss0, ss1, si0, si1, sd0, sd1):
    cid = lax.axis_index("c")
    sid = lax.axis_index("s")
    wid = cid * NS + sid
    base0 = wid * EPT
    dslot0 = didx.at[0]
    dslot1 = didx.at[1]

    @pl.loop(0, C1)
    def _(r):
        for j in range(NVR):
            ea0[r, pl.ds(j * LANES, LANES)] = jnp.zeros((LANES,), jnp.float32)

    @pl.loop(0, RPS // C1)
    def _(k):
        pltpu.sync_copy(ea0, agg_sh.at[pl.ds(sid * RPS + k * C1, C1)])

    plsc.subcore_barrier()

    def issue_idx(j, siv, si):
        jc = jnp.minimum(j, NF1 - 1)
        pltpu.async_copy(src_hbm.at[pl.ds(base0 + jc * C1, C1)], siv, si)

    def wait_idx(siv, si):
        pltpu.make_async_copy(src_hbm.at[pl.ds(0, C1)], siv, si).wait()

    def issue_didx(j, dslot, sd):
        jc = jnp.minimum(j, NF1 - 1)
        pltpu.async_copy(dst_hbm.at[pl.ds(base0 + jc * C1, C1)], dslot, sd)

    def wait_didx(dslot, sd):
        pltpu.make_async_copy(dst_hbm.at[pl.ds(0, C1)], dslot, sd).wait()

    def issue(j, siv, xs, ea, sg, se):
        jc = jnp.minimum(j, NF1 - 1)
        pltpu.async_copy(x_hbm.at[siv], xs, sg)
        pltpu.async_copy(ea_hbm.at[pl.ds(base0 + jc * C1, C1)], ea, se)

    def wait_loads(siv, xs, ea, sg, se):
        pltpu.make_async_copy(x_hbm.at[siv], xs, sg).wait()
        pltpu.make_async_copy(ea_hbm.at[pl.ds(0, C1)], ea, se).wait()

    def scatter(j, ea, dslot, ss):
        pltpu.async_copy(ea, agg_sh.at[dslot], ss, add=True)

    def wait_scatter(ea, ss):
        pltpu.make_async_copy(ea, agg_sh.at[dslot0], ss).wait()

    pltpu.sync_copy(dst_hbm.at[pl.ds(base0, C1)], dslot0)
    pltpu.sync_copy(dst_hbm.at[pl.ds(base0 + C1, C1)], dslot1)
    pltpu.sync_copy(src_hbm.at[pl.ds(base0, C1)], si0v)
    pltpu.sync_copy(src_hbm.at[pl.ds(base0 + C1, C1)], si1v)
    issue(0, si0v, xs0, ea0, sg0, se0)
    issue(1, si1v, xs1, ea1, sg1, se1)

    wait_loads(si0v, xs0, ea0, sg0, se0)
    issue_idx(2, si0v, si0)
    _mul_rows(C1, ea0, xs0)
    scatter(0, ea0, dslot0, ss0)

    wait_scatter(ea0, ss0)
    issue_didx(2, dslot0, sd0)
    wait_idx(si0v, si0)
    issue(2, si0v, xs0, ea0, sg0, se0)
    wait_loads(si1v, xs1, ea1, sg1, se1)
    issue_idx(3, si1v, si1)
    _mul_rows(C1, ea1, xs1)
    scatter(1, ea1, dslot1, ss1)

    @pl.loop(0, (NF1 - 2) // 2)
    def _(k):
        ja = 2 * k + 2
        wait_scatter(ea1, ss1)
        issue_didx(ja + 1, dslot1, sd1)
        wait_idx(si1v, si1)
        issue(ja + 1, si1v, xs1, ea1, sg1, se1)
        wait_loads(si0v, xs0, ea0, sg0, se0)
        issue_idx(ja + 2, si0v, si0)
        _mul_rows(C1, ea0, xs0)
        wait_didx(dslot0, sd0)
        scatter(ja, ea0, dslot0, ss0)

        jb = 2 * k + 3
        wait_scatter(ea0, ss0)
        issue_didx(jb + 1, dslot0, sd0)
        wait_idx(si0v, si0)
        issue(jb + 1, si0v, xs0, ea0, sg0, se0)
        wait_loads(si1v, xs1, ea1, sg1, se1)
        issue_idx(jb + 2, si1v, si1)
        _mul_rows(C1, ea1, xs1)
        wait_didx(dslot1, sd1)
        scatter(jb, ea1, dslot1, ss1)

    wait_scatter(ea1, ss1)
    wait_didx(dslot0, sd0)
    wait_idx(si1v, si1)
    wait_loads(si0v, xs0, ea0, sg0, se0)

    tbase = base0 + NF1 * C1
    pltpu.sync_copy(src_hbm.at[pl.ds(tbase, CT1)], si0v.at[pl.ds(0, CT1)])
    pltpu.sync_copy(dst_hbm.at[pl.ds(tbase, CT1)], didx_t.at[0])
    pltpu.sync_copy(x_hbm.at[si0v.at[pl.ds(0, CT1)]], xs0.at[pl.ds(0, CT1)])
    pltpu.sync_copy(ea_hbm.at[pl.ds(tbase, CT1)], ea0.at[pl.ds(0, CT1)])
    _mul_rows(CT1, ea0, xs0)
    pltpu.sync_copy(ea0.at[pl.ds(0, CT1)], agg_sh.at[didx_t.at[0]], add=True)

    plsc.subcore_barrier()

    pltpu.sync_copy(agg_sh.at[pl.ds(sid * RPS, RPS)],
                    out_hbm.at[pl.ds(cid * NPAD + sid * RPS, RPS)])


def _make_phase3(ks):
    return functools.partial(
        pl.kernel,
        out_type=jax.ShapeDtypeStruct((ES, D), jnp.float32),
        mesh=_mesh,
        scratch_types=[
            pltpu.VMEM((C3,), jnp.int32),
            pltpu.VMEM((C3,), jnp.int32),
            pltpu.VMEM((C3,), jnp.int32),
            pltpu.VMEM((C3,), jnp.int32),
            pltpu.VMEM((C3, D), jnp.float32),
            pltpu.VMEM((C3, D), jnp.float32),
            pltpu.VMEM((C3, D), jnp.float32),
            pltpu.VMEM((C3, D), jnp.float32),
            pltpu.VMEM_SHARED((NPAD, D), jnp.float32),
            pltpu.SemaphoreType.DMA,
            pltpu.SemaphoreType.DMA,
            pltpu.SemaphoreType.DMA,
            pltpu.SemaphoreType.DMA,
            pltpu.SemaphoreType.DMA,
            pltpu.SemaphoreType.DMA,
            pltpu.SemaphoreType.DMA,
            pltpu.SemaphoreType.DMA,
            pltpu.SemaphoreType.DMA,
            pltpu.SemaphoreType.DMA,
        ],
    )(functools.partial(_phase3_body, ks))


def _phase3_body(ks, na_hbm, src_hbm, dst_hbm, prod_hbm,
                 siv0, siv1, div0, div1, gs0, gs1, gd0, gd1, na_sh,
                 sg0, sg1, sdg0, sdg1, sw0, sw1, si0, si1, sdi0, sdi1):
    cid = lax.axis_index("c")
    sid = lax.axis_index("s")
    wid = cid * NS + sid

    @pl.when(sid < NS - 1)
    def _():
        pltpu.sync_copy(na_hbm.at[pl.ds(sid * RPS, RPS)],
                        na_sh.at[pl.ds(sid * RPS, RPS)])

    @pl.when(sid == NS - 1)
    def _():
        pltpu.sync_copy(na_hbm.at[pl.ds((NS - 1) * RPS, N - (NS - 1) * RPS)],
                        na_sh.at[pl.ds((NS - 1) * RPS, N - (NS - 1) * RPS)])

    def issue_idx(j, siv, div, si, sdi):
        jc = jnp.minimum(j, NCHUNK3 - 1)
        pltpu.async_copy(src_hbm.at[ks, wid, jc], siv, si)
        pltpu.async_copy(dst_hbm.at[ks, wid, jc], div, sdi)

    def wait_idx(siv, div, si, sdi):
        pltpu.make_async_copy(src_hbm.at[0, 0, 0], siv, si).wait()
        pltpu.make_async_copy(dst_hbm.at[0, 0, 0], div, sdi).wait()

    def issue(siv, div, gs, gd, sg, sdg):
        pltpu.async_copy(na_sh.at[siv], gs, sg)
        pltpu.async_copy(na_sh.at[div], gd, sdg)

    def wait_gathers(siv, div, gs, gd, sg, sdg):
        pltpu.make_async_copy(na_sh.at[siv], gs, sg).wait()
        pltpu.make_async_copy(na_sh.at[div], gd, sdg).wait()

    def write(j, gs, sw):
        pltpu.async_copy(gs, prod_hbm.at[pl.ds(wid * EPTS + j * C3, C3)], sw)

    def wait_write(gs, sw):
        pltpu.make_async_copy(gs, prod_hbm.at[pl.ds(0, C3)], sw).wait()

    pltpu.sync_copy(src_hbm.at[ks, wid, 0], siv0)
    pltpu.sync_copy(dst_hbm.at[ks, wid, 0], div0)
    pltpu.sync_copy(src_hbm.at[ks, wid, 1], siv1)
    pltpu.sync_copy(dst_hbm.at[ks, wid, 1], div1)
    plsc.subcore_barrier()

    issue(siv0, div0, gs0, gd0, sg0, sdg0)
    issue(siv1, div1, gs1, gd1, sg1, sdg1)

    wait_gathers(siv0, div0, gs0, gd0, sg0, sdg0)
    issue_idx(2, siv0, div0, si0, sdi0)
    _mul_rows(C3, gs0, gd0)
    write(0, gs0, sw0)

    wait_write(gs0, sw0)
    wait_idx(siv0, div0, si0, sdi0)
    issue(siv0, div0, gs0, gd0, sg0, sdg0)
    wait_gathers(siv1, div1, gs1, gd1, sg1, sdg1)
    issue_idx(3, siv1, div1, si1, sdi1)
    _mul_rows(C3, gs1, gd1)
    write(1, gs1, sw1)

    @pl.loop(0, (NCHUNK3 - 3) // 2)
    def _(k):
        ja = 2 * k + 2
        wait_write(gs1, sw1)
        wait_idx(siv1, div1, si1, sdi1)
        issue(siv1, div1, gs1, gd1, sg1, sdg1)
        wait_gathers(siv0, div0, gs0, gd0, sg0, sdg0)
        issue_idx(ja + 2, siv0, div0, si0, sdi0)
        _mul_rows(C3, gs0, gd0)
        write(ja, gs0, sw0)

        jb = 2 * k + 3
        wait_write(gs0, sw0)
        wait_idx(siv0, div0, si0, sdi0)
        issue(siv0, div0, gs0, gd0, sg0, sdg0)
        wait_gathers(siv1, div1, gs1, gd1, sg1, sdg1)
        issue_idx(jb + 2, siv1, div1, si1, sdi1)
        _mul_rows(C3, gs1, gd1)
        write(jb, gs1, sw1)

    wait_write(gs1, sw1)
    wait_gathers(siv0, div0, gs0, gd0, sg0, sdg0)
    _mul_rows(C3, gs0, gd0)
    write(NCHUNK3 - 1, gs0, sw0)
    wait_write(gs0, sw0)
    wait_idx(siv1, div1, si1, sdi1)


def _node_softmax_body(p_ref, x_ref, o_ref):
    a = p_ref[0:N, :] + p_ref[NPAD:NPAD + N, :] + x_ref[...]
    m = jnp.max(a, axis=-1, keepdims=True)
    e = jnp.exp(a - m)
    o_ref[...] = e / jnp.sum(e, axis=-1, keepdims=True)


def _edge_softmax_first(prod_ref, ea_ref, o_ref):
    t = prod_ref[...] + ea_ref[...]
    m = jnp.max(t, axis=-1, keepdims=True)
    e = jnp.exp(t - m)
    o_ref[...] = e / jnp.sum(e, axis=-1, keepdims=True)


def _edge_softmax_next(buf_ref, prod_ref, ea_ref, o_ref):
    del buf_ref
    t = prod_ref[...] + ea_ref[...]
    m = jnp.max(t, axis=-1, keepdims=True)
    e = jnp.exp(t - m)
    o_ref[...] = e / jnp.sum(e, axis=-1, keepdims=True)


_BE = 2000
_BPS = ES // _BE

_PHASE3 = [_make_phase3(k) for k in range(KS)]


def kernel(x, edge_attr, edge_index):
    src = edge_index[0].astype(jnp.int32)
    dst = edge_index[1].astype(jnp.int32)

    partials = _phase1(x, edge_attr, src, dst)

    node_att = pl.pallas_call(
        _node_softmax_body,
        out_shape=jax.ShapeDtypeStruct((N, D), jnp.float32),
    )(partials, x)

    src3 = src.reshape(KS, NW, NCHUNK3, C3)
    dst3 = dst.reshape(KS, NW, NCHUNK3, C3)

    prods = [_PHASE3[k](node_att, src3, dst3) for k in range(KS)]

    edge_att_new = pl.pallas_call(
        _edge_softmax_first,
        grid=(_BPS,),
        in_specs=[pl.BlockSpec((_BE, D), lambda i: (i, 0))] * 2,
        out_specs=pl.BlockSpec((_BE, D), lambda i: (i, 0)),
        out_shape=jax.ShapeDtypeStruct((E, D), jnp.float32),
    )(prods[0], edge_attr)

    for k in range(1, KS):
        off = k * _BPS
        edge_att_new = pl.pallas_call(
            _edge_softmax_next,
            grid=(_BPS,),
            in_specs=[
                pl.BlockSpec(memory_space=pl.ANY),
                pl.BlockSpec((_BE, D), lambda i: (i, 0)),
                pl.BlockSpec((_BE, D), lambda i, off=off: (i + off, 0)),
            ],
            out_specs=pl.BlockSpec((_BE, D), lambda i, off=off: (i + off, 0)),
            out_shape=jax.ShapeDtypeStruct((E, D), jnp.float32),
            input_output_aliases={0: 0},
        )(edge_att_new, prods[k], edge_attr)

    return node_att, edge_att_new


# speedup vs baseline: 6.6748x; 1.0244x over previous
import functools

import jax
import jax.numpy as jnp
from jax import lax
from jax.experimental import pallas as pl
from jax.experimental.pallas import tpu as pltpu
from jax.experimental.pallas import tpu_sc as plsc

N = 10000
E = 320000
D = 128
NC = 2
NS = 16
NW = NC * NS
EPT = E // NW
C1 = 64
NF1 = EPT // C1
CT1 = EPT - NF1 * C1
C3 = 64
KS = 2
ES = E // KS
EPTS = ES // NW
NF3 = EPTS // C3
CT3 = EPTS - NF3 * C3
NPAD = 10240
RPS = NPAD // NS
LANES = 16
NVR = D // LANES

_mesh = plsc.VectorSubcoreMesh(core_axis_name="c", subcore_axis_name="s")


def _mul_rows(nrows, acc, other):
    @pl.loop(0, nrows)
    def _(r):
        for j in range(NVR):
            sl = pl.ds(j * LANES, LANES)
            acc[r, sl] = acc[r, sl] * other[r, sl]


@functools.partial(
    pl.kernel,
    out_type=jax.ShapeDtypeStruct((NC * NPAD, D), jnp.float32),
    mesh=_mesh,
    scratch_types=[
        pltpu.VMEM((C1,), jnp.int32),
        pltpu.VMEM((C1,), jnp.int32),
        pltpu.VMEM((2, C1), jnp.int32),
        pltpu.VMEM((1, CT1), jnp.int32),
        pltpu.VMEM((C1, D), jnp.float32),
        pltpu.VMEM((C1, D), jnp.float32),
        pltpu.VMEM((C1, D), jnp.float32),
        pltpu.VMEM((C1, D), jnp.float32),
        pltpu.VMEM_SHARED((NPAD, D), jnp.float32),
        pltpu.SemaphoreType.DMA,
        pltpu.SemaphoreType.DMA,
        pltpu.SemaphoreType.DMA,
        pltpu.SemaphoreType.DMA,
        pltpu.SemaphoreType.DMA,
        pltpu.SemaphoreType.DMA,
        pltpu.SemaphoreType.DMA,
        pltpu.SemaphoreType.DMA,
        pltpu.SemaphoreType.DMA,
        pltpu.SemaphoreType.DMA,
    ],
)
def _phase1(x_hbm, ea_hbm, src_hbm, dst_hbm, out_hbm,
            si0v, si1v, didx, didx_t, xs0, xs1, ea0, ea1, agg_sh,
            sg0, sg1, se0, se1, ss0, ss1, si0, si1, sd0, sd1):
    cid = lax.axis_index("c")
    sid = lax.axis_index("s")
    wid = cid * NS + sid
    base0 = wid * EPT
    dslot0 = didx.at[0]
    dslot1 = didx.at[1]

    @pl.loop(0, C1)
    def _(r):
        for j in range(NVR):
            ea0[r, pl.ds(j * LANES, LANES)] = jnp.zeros((LANES,), jnp.float32)

    @pl.loop(0, RPS // C1)
    def _(k):
        pltpu.sync_copy(ea0, agg_sh.at[pl.ds(sid * RPS + k * C1, C1)])

    plsc.subcore_barrier()

    def issue_idx(j, siv, si):
        jc = jnp.minimum(j, NF1 - 1)
        pltpu.async_copy(src_hbm.at[pl.ds(base0 + jc * C1, C1)], siv, si)

    def wait_idx(siv, si):
        pltpu.make_async_copy(src_hbm.at[pl.ds(0, C1)], siv, si).wait()

    def issue_didx(j, dslot, sd):
        jc = jnp.minimum(j, NF1 - 1)
        pltpu.async_copy(dst_hbm.at[pl.ds(base0 + jc * C1, C1)], dslot, sd)

    def wait_didx(dslot, sd):
        pltpu.make_async_copy(dst_hbm.at[pl.ds(0, C1)], dslot, sd).wait()

    def issue(j, siv, xs, ea, sg, se):
        jc = jnp.minimum(j, NF1 - 1)
        pltpu.async_copy(x_hbm.at[siv], xs, sg)
        pltpu.async_copy(ea_hbm.at[pl.ds(base0 + jc * C1, C1)], ea, se)

    def wait_loads(siv, xs, ea, sg, se):
        pltpu.make_async_copy(x_hbm.at[siv], xs, sg).wait()
        pltpu.make_async_copy(ea_hbm.at[pl.ds(0, C1)], ea, se).wait()

    def scatter(j, ea, dslot, ss):
        pltpu.async_copy(ea, agg_sh.at[dslot], ss, add=True)

    def wait_scatter(ea, ss):
        pltpu.make_async_copy(ea, agg_sh.at[dslot0], ss).wait()

    pltpu.sync_copy(dst_hbm.at[pl.ds(base0, C1)], dslot0)
    pltpu.sync_copy(dst_hbm.at[pl.ds(base0 + C1, C1)], dslot1)
    pltpu.sync_copy(src_hbm.at[pl.ds(base0, C1)], si0v)
    pltpu.sync_copy(src_hbm.at[pl.ds(base0 + C1, C1)], si1v)
    issue(0, si0v, xs0, ea0, sg0, se0)
    issue(1, si1v, xs1, ea1, sg1, se1)

    wait_loads(si0v, xs0, ea0, sg0, se0)
    issue_idx(2, si0v, si0)
    _mul_rows(C1, ea0, xs0)
    scatter(0, ea0, dslot0, ss0)

    wait_scatter(ea0, ss0)
    issue_didx(2, dslot0, sd0)
    wait_idx(si0v, si0)
    issue(2, si0v, xs0, ea0, sg0, se0)
    wait_loads(si1v, xs1, ea1, sg1, se1)
    issue_idx(3, si1v, si1)
    _mul_rows(C1, ea1, xs1)
    scatter(1, ea1, dslot1, ss1)

    @pl.loop(0, (NF1 - 2) // 2)
    def _(k):
        ja = 2 * k + 2
        wait_scatter(ea1, ss1)
        issue_didx(ja + 1, dslot1, sd1)
        wait_idx(si1v, si1)
        issue(ja + 1, si1v, xs1, ea1, sg1, se1)
        wait_loads(si0v, xs0, ea0, sg0, se0)
        issue_idx(ja + 2, si0v, si0)
        _mul_rows(C1, ea0, xs0)
        wait_didx(dslot0, sd0)
        scatter(ja, ea0, dslot0, ss0)

        jb = 2 * k + 3
        wait_scatter(ea0, ss0)
        issue_didx(jb + 1, dslot0, sd0)
        wait_idx(si0v, si0)
        issue(jb + 1, si0v, xs0, ea0, sg0, se0)
        wait_loads(si1v, xs1, ea1, sg1, se1)
        issue_idx(jb + 2, si1v, si1)
        _mul_rows(C1, ea1, xs1)
        wait_didx(dslot1, sd1)
        scatter(jb, ea1, dslot1, ss1)

    wait_scatter(ea1, ss1)
    wait_didx(dslot0, sd0)
    wait_idx(si1v, si1)
    wait_loads(si0v, xs0, ea0, sg0, se0)

    tbase = base0 + NF1 * C1
    pltpu.sync_copy(src_hbm.at[pl.ds(tbase, CT1)], si0v.at[pl.ds(0, CT1)])
    pltpu.sync_copy(dst_hbm.at[pl.ds(tbase, CT1)], didx_t.at[0])
    pltpu.sync_copy(x_hbm.at[si0v.at[pl.ds(0, CT1)]], xs0.at[pl.ds(0, CT1)])
    pltpu.sync_copy(ea_hbm.at[pl.ds(tbase, CT1)], ea0.at[pl.ds(0, CT1)])
    _mul_rows(CT1, ea0, xs0)
    pltpu.sync_copy(ea0.at[pl.ds(0, CT1)], agg_sh.at[didx_t.at[0]], add=True)

    plsc.subcore_barrier()

    pltpu.sync_copy(agg_sh.at[pl.ds(sid * RPS, RPS)],
                    out_hbm.at[pl.ds(cid * NPAD + sid * RPS, RPS)])


def _make_phase3(ks):
    return functools.partial(
        pl.kernel,
        out_type=jax.ShapeDtypeStruct((ES, D), jnp.float32),
        mesh=_mesh,
        scratch_types=[
            pltpu.VMEM((C3,), jnp.int32),
            pltpu.VMEM((C3,), jnp.int32),
            pltpu.VMEM((C3,), jnp.int32),
            pltpu.VMEM((C3,), jnp.int32),
            pltpu.VMEM((C3, D), jnp.float32),
            pltpu.VMEM((C3, D), jnp.float32),
            pltpu.VMEM((C3, D), jnp.float32),
            pltpu.VMEM((C3, D), jnp.float32),
            pltpu.VMEM_SHARED((NPAD, D), jnp.float32),
            pltpu.SemaphoreType.DMA,
            pltpu.SemaphoreType.DMA,
            pltpu.SemaphoreType.DMA,
            pltpu.SemaphoreType.DMA,
            pltpu.SemaphoreType.DMA,
            pltpu.SemaphoreType.DMA,
            pltpu.SemaphoreType.DMA,
            pltpu.SemaphoreType.DMA,
            pltpu.SemaphoreType.DMA,
            pltpu.SemaphoreType.DMA,
        ],
    )(functools.partial(_phase3_body, ks))


def _phase3_body(ks, na_hbm, src_hbm, dst_hbm, prod_hbm,
                 siv0, siv1, div0, div1, gs0, gs1, gd0, gd1, na_sh,
                 sg0, sg1, sdg0, sdg1, sw0, sw1, si0, si1, sdi0, sdi1):
    cid = lax.axis_index("c")
    sid = lax.axis_index("s")
    wid = cid * NS + sid
    sbase = ks * ES + wid * EPTS

    @pl.when(sid < NS - 1)
    def _():
        pltpu.sync_copy(na_hbm.at[pl.ds(sid * RPS, RPS)],
                        na_sh.at[pl.ds(sid * RPS, RPS)])

    @pl.when(sid == NS - 1)
    def _():
        pltpu.sync_copy(na_hbm.at[pl.ds((NS - 1) * RPS, N - (NS - 1) * RPS)],
                        na_sh.at[pl.ds((NS - 1) * RPS, N - (NS - 1) * RPS)])

    def issue_idx(j, siv, div, si, sdi):
        jc = jnp.minimum(j, NF3 - 1)
        pltpu.async_copy(src_hbm.at[pl.ds(sbase + jc * C3, C3)], siv, si)
        pltpu.async_copy(dst_hbm.at[pl.ds(sbase + jc * C3, C3)], div, sdi)

    def wait_idx(siv, div, si, sdi):
        pltpu.make_async_copy(src_hbm.at[pl.ds(0, C3)], siv, si).wait()
        pltpu.make_async_copy(dst_hbm.at[pl.ds(0, C3)], div, sdi).wait()

    def issue(siv, div, gs, gd, sg, sdg):
        pltpu.async_copy(na_sh.at[siv], gs, sg)
        pltpu.async_copy(na_sh.at[div], gd, sdg)

    def wait_gathers(siv, div, gs, gd, sg, sdg):
        pltpu.make_async_copy(na_sh.at[siv], gs, sg).wait()
        pltpu.make_async_copy(na_sh.at[div], gd, sdg).wait()

    def write(j, gs, sw):
        pltpu.async_copy(gs, prod_hbm.at[pl.ds(wid * EPTS + j * C3, C3)], sw)

    def wait_write(gs, sw):
        pltpu.make_async_copy(gs, prod_hbm.at[pl.ds(0, C3)], sw).wait()

    pltpu.sync_copy(src_hbm.at[pl.ds(sbase, C3)], siv0)
    pltpu.sync_copy(dst_hbm.at[pl.ds(sbase, C3)], div0)
    pltpu.sync_copy(src_hbm.at[pl.ds(sbase + C3, C3)], siv1)
    pltpu.sync_copy(dst_hbm.at[pl.ds(sbase + C3, C3)], div1)
    plsc.subcore_barrier()

    issue(siv0, div0, gs0, gd0, sg0, sdg0)
    issue(siv1, div1, gs1, gd1, sg1, sdg1)

    wait_gathers(siv0, div0, gs0, gd0, sg0, sdg0)
    issue_idx(2, siv0, div0, si0, sdi0)
    _mul_rows(C3, gs0, gd0)
    write(0, gs0, sw0)

    wait_write(gs0, sw0)
    wait_idx(siv0, div0, si0, sdi0)
    issue(siv0, div0, gs0, gd0, sg0, sdg0)
    wait_gathers(siv1, div1, gs1, gd1, sg1, sdg1)
    issue_idx(3, siv1, div1, si1, sdi1)
    _mul_rows(C3, gs1, gd1)
    write(1, gs1, sw1)

    @pl.loop(0, (NF3 - 2) // 2)
    def _(k):
        ja = 2 * k + 2
        wait_write(gs1, sw1)
        wait_idx(siv1, div1, si1, sdi1)
        issue(siv1, div1, gs1, gd1, sg1, sdg1)
        wait_gathers(siv0, div0, gs0, gd0, sg0, sdg0)
        issue_idx(ja + 2, siv0, div0, si0, sdi0)
        _mul_rows(C3, gs0, gd0)
        write(ja, gs0, sw0)

        jb = 2 * k + 3
        wait_write(gs0, sw0)
        wait_idx(siv0, div0, si0, sdi0)
        issue(siv0, div0, gs0, gd0, sg0, sdg0)
        wait_gathers(siv1, div1, gs1, gd1, sg1, sdg1)
        issue_idx(jb + 2, siv1, div1, si1, sdi1)
        _mul_rows(C3, gs1, gd1)
        write(jb, gs1, sw1)

    wait_write(gs1, sw1)
    wait_gathers(siv0, div0, gs0, gd0, sg0, sdg0)
    wait_idx(siv1, div1, si1, sdi1)

    tbase = sbase + NF3 * C3
    pltpu.sync_copy(src_hbm.at[pl.ds(tbase, CT3)], siv0.at[pl.ds(0, CT3)])
    pltpu.sync_copy(dst_hbm.at[pl.ds(tbase, CT3)], div0.at[pl.ds(0, CT3)])
    pltpu.sync_copy(na_sh.at[siv0.at[pl.ds(0, CT3)]], gs0.at[pl.ds(0, CT3)])
    pltpu.sync_copy(na_sh.at[div0.at[pl.ds(0, CT3)]], gd0.at[pl.ds(0, CT3)])
    _mul_rows(CT3, gs0, gd0)
    pltpu.sync_copy(gs0.at[pl.ds(0, CT3)],
                    prod_hbm.at[pl.ds(wid * EPTS + NF3 * C3, CT3)])


def _node_softmax_body(p_ref, x_ref, o_ref):
    a = p_ref[0:N, :] + p_ref[NPAD:NPAD + N, :] + x_ref[...]
    m = jnp.max(a, axis=-1, keepdims=True)
    e = jnp.exp(a - m)
    o_ref[...] = e / jnp.sum(e, axis=-1, keepdims=True)


def _edge_softmax_first(prod_ref, ea_ref, o_ref):
    t = prod_ref[...] + ea_ref[...]
    m = jnp.max(t, axis=-1, keepdims=True)
    e = jnp.exp(t - m)
    o_ref[...] = e / jnp.sum(e, axis=-1, keepdims=True)


def _edge_softmax_next(buf_ref, prod_ref, ea_ref, o_ref):
    del buf_ref
    t = prod_ref[...] + ea_ref[...]
    m = jnp.max(t, axis=-1, keepdims=True)
    e = jnp.exp(t - m)
    o_ref[...] = e / jnp.sum(e, axis=-1, keepdims=True)


_BE = 2000
_BPS = ES // _BE

_PHASE3 = [_make_phase3(k) for k in range(KS)]


def kernel(x, edge_attr, edge_index):
    src = edge_index[0].astype(jnp.int32)
    dst = edge_index[1].astype(jnp.int32)

    partials = _phase1(x, edge_attr, src, dst)

    node_att = pl.pallas_call(
        _node_softmax_body,
        out_shape=jax.ShapeDtypeStruct((N, D), jnp.float32),
    )(partials, x)

    prods = [_PHASE3[k](node_att, src, dst) for k in range(KS)]

    edge_att_new = pl.pallas_call(
        _edge_softmax_first,
        grid=(_BPS,),
        in_specs=[pl.BlockSpec((_BE, D), lambda i: (i, 0))] * 2,
        out_specs=pl.BlockSpec((_BE, D), lambda i: (i, 0)),
        out_shape=jax.ShapeDtypeStruct((E, D), jnp.float32),
    )(prods[0], edge_attr)

    for k in range(1, KS):
        off = k * _BPS
        edge_att_new = pl.pallas_call(
            _edge_softmax_next,
            grid=(_BPS,),
            in_specs=[
                pl.BlockSpec(memory_space=pl.ANY),
                pl.BlockSpec((_BE, D), lambda i: (i, 0)),
                pl.BlockSpec((_BE, D), lambda i, off=off: (i + off, 0)),
            ],
            out_specs=pl.BlockSpec((_BE, D), lambda i, off=off: (i + off, 0)),
            out_shape=jax.ShapeDtypeStruct((E, D), jnp.float32),
            input_output_aliases={0: 0},
        )(edge_att_new, prods[k], edge_attr)

    return node_att, edge_att_new
